# baseline (speedup 1.0000x reference)
import jax
import jax.numpy as jnp
from jax import lax
from jax.experimental import pallas as pl
from jax.experimental.pallas import tpu as pltpu

B, S, K, D = 4096, 20, 64, 32
BB = 512
NBLK = B // BB


def _body(s_ref, m_ref, k_ref, uvt_ref, wt_ref, o_ref,
          h_ref, kt_ref, st_ref, mt_ref):
    f32 = jnp.float32
    uvt = uvt_ref[...]
    wt = wt_ref[...]

    kt_all = k_ref[...].T
    kt_ref[...] = kt_all
    h_ref[...] = kt_all
    st_all = s_ref[...].T
    mt_all = m_ref[...].T
    for t in range(S):
        st_ref[t] = st_all[t * D:(t + 1) * D, :]
        mt_ref[t] = mt_all[t:t + 1, :]

    def step(t, carry):
        s_t = st_ref[pl.ds(t, 1)].reshape(D, BB)
        m_t = mt_ref[pl.ds(t, 1)].reshape(1, BB) > 0.5
        sw_t = jnp.dot(wt, s_t, preferred_element_type=f32)
        for k in range(K):
            r = pl.ds(k * D, D)
            h = h_ref[r, :]
            kt = kt_ref[r, :]
            g_half = 0.5 * jnp.sum(s_t * (h + kt), axis=0, keepdims=True)
            g4 = 0.25 * jnp.tanh(g_half) + 0.25
            huv = jnp.dot(uvt, jnp.concatenate([h, kt], axis=0),
                          preferred_element_type=f32)
            z = huv + sw_t
            upd = h + g4 * (jnp.tanh(z) + 1.0)
            sq = jnp.sum(upd * upd, axis=0, keepdims=True)
            upd = upd * lax.rsqrt(jnp.maximum(sq, 1e-12))
            h_ref[r, :] = jnp.where(m_t, upd, h)
        return carry

    lax.fori_loop(0, S, step, 0)
    o_ref[...] = h_ref[...].T


@jax.jit
def kernel(encoded_sents, mask, keys, U, V, W):
    f32 = jnp.float32
    grid = (NBLK,)
    out = pl.pallas_call(
        _body,
        grid=grid,
        in_specs=[
            pl.BlockSpec((BB, S * D), lambda i: (i, 0)),
            pl.BlockSpec((BB, S), lambda i: (i, 0)),
            pl.BlockSpec((BB, K * D), lambda i: (i, 0)),
            pl.BlockSpec((D, 2 * D), lambda i: (0, 0)),
            pl.BlockSpec((D, D), lambda i: (0, 0)),
        ],
        out_specs=pl.BlockSpec((BB, K * D), lambda i: (i, 0)),
        out_shape=jax.ShapeDtypeStruct((B, K * D), f32),
        scratch_shapes=[
            pltpu.VMEM((K * D, BB), f32),
            pltpu.VMEM((K * D, BB), f32),
            pltpu.VMEM((S, D, BB), f32),
            pltpu.VMEM((S, 1, BB), f32),
        ],
    )(encoded_sents.reshape(B, S * D), mask.astype(f32),
      keys.reshape(B, K * D),
      0.5 * jnp.concatenate([U.T, V.T], axis=1), 0.5 * W.T)
    return out.reshape(B, K, D)


# speedup vs baseline: 1.0873x; 1.0873x over previous
import jax
import jax.numpy as jnp
from jax import lax
from jax.experimental import pallas as pl
from jax.experimental.pallas import tpu as pltpu

B, S, K, D = 4096, 20, 64, 32
BB = 256
NBLK = B // BB


def _body(s_ref, m_ref, k_ref, uvt_ref, wt_ref, o_ref,
          h_ref, kt_ref, st_ref, mt_ref):
    f32 = jnp.float32
    uvt = uvt_ref[...]
    wt = wt_ref[...]

    kt_all = k_ref[...].T
    kt_ref[...] = kt_all
    h_ref[...] = kt_all
    st_all = s_ref[...].T
    mt_all = m_ref[...].T
    for t in range(S):
        st_ref[t] = st_all[t * D:(t + 1) * D, :]
        mt_ref[t] = mt_all[t:t + 1, :]

    def one_step(t):
        s_t = st_ref[pl.ds(t, 1)].reshape(D, BB)
        m_t = mt_ref[pl.ds(t, 1)].reshape(1, BB) > 0.5
        sw_t = jnp.dot(wt, s_t, preferred_element_type=f32)
        for k in range(K):
            r = pl.ds(k * D, D)
            h = h_ref[r, :]
            kt = kt_ref[r, :]
            g_half = 0.5 * jnp.sum(s_t * (h + kt), axis=0, keepdims=True)
            g4 = 0.25 * jnp.tanh(g_half) + 0.25
            huv = jnp.dot(uvt, jnp.concatenate([h, kt], axis=0),
                          preferred_element_type=f32)
            z = huv + sw_t
            upd = h + g4 * (jnp.tanh(z) + 1.0)
            sq = jnp.sum(upd * upd, axis=0, keepdims=True)
            upd = upd * lax.rsqrt(jnp.maximum(sq, 1e-12))
            h_ref[r, :] = jnp.where(m_t, upd, h)

    def step2(i, carry):
        one_step(2 * i)
        one_step(2 * i + 1)
        return carry

    lax.fori_loop(0, S // 2, step2, 0)
    o_ref[...] = h_ref[...].T


@jax.jit
def kernel(encoded_sents, mask, keys, U, V, W):
    f32 = jnp.float32
    grid = (NBLK,)
    out = pl.pallas_call(
        _body,
        grid=grid,
        in_specs=[
            pl.BlockSpec((BB, S * D), lambda i: (i, 0)),
            pl.BlockSpec((BB, S), lambda i: (i, 0)),
            pl.BlockSpec((BB, K * D), lambda i: (i, 0)),
            pl.BlockSpec((D, 2 * D), lambda i: (0, 0)),
            pl.BlockSpec((D, D), lambda i: (0, 0)),
        ],
        out_specs=pl.BlockSpec((BB, K * D), lambda i: (i, 0)),
        out_shape=jax.ShapeDtypeStruct((B, K * D), f32),
        scratch_shapes=[
            pltpu.VMEM((K * D, BB), f32),
            pltpu.VMEM((K * D, BB), f32),
            pltpu.VMEM((S, D, BB), f32),
            pltpu.VMEM((S, 1, BB), f32),
        ],
    )(encoded_sents.reshape(B, S * D), mask.astype(f32),
      keys.reshape(B, K * D),
      0.5 * jnp.concatenate([U.T, V.T], axis=1), 0.5 * W.T)
    return out.reshape(B, K, D)


# speedup vs baseline: 1.0948x; 1.0069x over previous
import jax
import jax.numpy as jnp
from jax import lax
from jax.experimental import pallas as pl
from jax.experimental.pallas import tpu as pltpu

B, S, K, D = 4096, 20, 64, 32
BB = 256
NBLK = B // BB


def _body(s_ref, m_ref, k_ref, uvt_ref, wt_ref, o_ref,
          h_ref, kt_ref, st_ref, mt_ref):
    f32 = jnp.float32
    uvt = uvt_ref[...]
    wt = wt_ref[...]

    kt_all = k_ref[...].T
    kt_ref[...] = kt_all
    h_ref[...] = kt_all
    st_all = s_ref[...].T
    mt_all = m_ref[...].T
    for t in range(S):
        st_ref[t] = st_all[t * D:(t + 1) * D, :]
        mt_ref[t] = mt_all[t:t + 1, :]

    def one_step(t):
        s_t = st_ref[pl.ds(t, 1)].reshape(D, BB)
        m_t = mt_ref[pl.ds(t, 1)].reshape(1, BB) > 0.5
        sw_t = jnp.dot(wt, s_t, preferred_element_type=f32)
        for k in range(K):
            r = pl.ds(k * D, D)
            h = h_ref[r, :]
            kt = kt_ref[r, :]
            g_half = 0.5 * jnp.sum(s_t * (h + kt), axis=0, keepdims=True)
            g4 = 0.25 * jnp.tanh(g_half) + 0.25
            huv = jnp.dot(uvt, jnp.concatenate([h, kt], axis=0),
                          preferred_element_type=f32)
            z = huv + sw_t
            upd = h + g4 * (jnp.tanh(z) + 1.0)
            sq = jnp.sum(upd * upd, axis=0, keepdims=True)
            upd = upd * lax.rsqrt(jnp.maximum(sq, 1e-12))
            h_ref[r, :] = jnp.where(m_t, upd, h)

    def step4(i, carry):
        one_step(4 * i)
        one_step(4 * i + 1)
        one_step(4 * i + 2)
        one_step(4 * i + 3)
        return carry

    lax.fori_loop(0, S // 4, step4, 0)
    o_ref[...] = h_ref[...].T


@jax.jit
def kernel(encoded_sents, mask, keys, U, V, W):
    f32 = jnp.float32
    grid = (NBLK,)
    out = pl.pallas_call(
        _body,
        grid=grid,
        in_specs=[
            pl.BlockSpec((BB, S * D), lambda i: (i, 0)),
            pl.BlockSpec((BB, S), lambda i: (i, 0)),
            pl.BlockSpec((BB, K * D), lambda i: (i, 0)),
            pl.BlockSpec((D, 2 * D), lambda i: (0, 0)),
            pl.BlockSpec((D, D), lambda i: (0, 0)),
        ],
        out_specs=pl.BlockSpec((BB, K * D), lambda i: (i, 0)),
        out_shape=jax.ShapeDtypeStruct((B, K * D), f32),
        scratch_shapes=[
            pltpu.VMEM((K * D, BB), f32),
            pltpu.VMEM((K * D, BB), f32),
            pltpu.VMEM((S, D, BB), f32),
            pltpu.VMEM((S, 1, BB), f32),
        ],
    )(encoded_sents.reshape(B, S * D), mask.astype(f32),
      keys.reshape(B, K * D),
      0.5 * jnp.concatenate([U.T, V.T], axis=1), 0.5 * W.T)
    return out.reshape(B, K, D)
